# baseline (speedup 1.0000x reference)
import functools

import jax
import jax.numpy as jnp
from jax import lax
from jax.experimental import pallas as pl
from jax.experimental.pallas import tpu as pltpu
from jax.experimental.pallas import tpu_sc as plsc

B, C, H, W = 64, 192, 32, 32
HW = H * W
ROWS = B * C

_info = plsc.get_sparse_core_info()
_NC, _NS, _L = _info.num_cores, _info.num_subcores, _info.num_lanes
_NW = _NC * _NS
_BPW = B // _NW
CHUNK = 96


def _shuffle_body(x_hbm, perm_hbm, out_hbm, perm_v, idx_v, rows_v, sem):
    wid = lax.axis_index("s") * _NC + lax.axis_index("c")
    pltpu.sync_copy(perm_hbm, perm_v)
    for bi in range(_BPW):
        base = (wid * _BPW + bi) * C
        for c0 in range(0, C, CHUNK):
            for i in range(CHUNK // _L):
                idx_v[pl.ds(i * _L, _L)] = perm_v[pl.ds(c0 + i * _L, _L)] + base
            pltpu.async_copy(x_hbm.at[idx_v], rows_v, sem).wait()
            pltpu.sync_copy(rows_v, out_hbm.at[pl.ds(base + c0, CHUNK)])


_shuffle = functools.partial(
    pl.kernel,
    mesh=plsc.VectorSubcoreMesh(core_axis_name="c", subcore_axis_name="s"),
    out_type=jax.ShapeDtypeStruct((ROWS, HW), jnp.float32),
    scratch_types=[
        pltpu.VMEM((C,), jnp.int32),
        pltpu.VMEM((CHUNK,), jnp.int32),
        pltpu.VMEM((CHUNK, HW), jnp.float32),
        pltpu.SemaphoreType.DMA,
    ],
)(_shuffle_body)


def kernel(x, objective, indices):
    y = _shuffle(x.reshape(ROWS, HW), indices)
    return y.reshape(B, C, H, W), objective


# speedup vs baseline: 1.0014x; 1.0014x over previous
import functools

import jax
import jax.numpy as jnp
from jax import lax
from jax.experimental import pallas as pl
from jax.experimental.pallas import tpu as pltpu
from jax.experimental.pallas import tpu_sc as plsc

B, C, H, W = 64, 192, 32, 32
HW = H * W
ROWS = B * C

_info = plsc.get_sparse_core_info()
_NC, _NS, _L = _info.num_cores, _info.num_subcores, _info.num_lanes
_NW = _NC * _NS
_BPW = B // _NW
CHUNK = 48
_CPB = C // CHUNK
_NCH = _BPW * _CPB


def _shuffle_body(x_hbm, perm_hbm, out_hbm, perm_v, idx_v, rows_v,
                  gs0, gs1, ss0, ss1):
    wid = lax.axis_index("s") * _NC + lax.axis_index("c")
    pltpu.sync_copy(perm_hbm, perm_v)
    gsem = (gs0, gs1)
    ssem = (ss0, ss1)

    def fill_idx(ch, buf):
        bi, c0i = divmod(ch, _CPB)
        c0 = c0i * CHUNK
        bbase = (wid * _BPW + bi) * C
        for i in range(CHUNK // _L):
            idx_v[buf, pl.ds(i * _L, _L)] = (
                perm_v[pl.ds(c0 + i * _L, _L)] + bbase)
        return bbase + c0

    def start_gather(ch):
        buf = ch % 2
        off = fill_idx(ch, buf)
        cp = pltpu.async_copy(x_hbm.at[idx_v.at[buf]], rows_v.at[buf],
                              gsem[buf])
        return off, cp

    offs = [None] * _NCH
    gath = [None] * _NCH
    scat = [None] * _NCH
    offs[0], gath[0] = start_gather(0)
    for ch in range(_NCH):
        buf = ch % 2
        gath[ch].wait()
        scat[ch] = pltpu.async_copy(
            rows_v.at[buf], out_hbm.at[pl.ds(offs[ch], CHUNK)], ssem[buf])
        if ch + 1 < _NCH:
            if ch >= 1:
                scat[ch - 1].wait()
            offs[ch + 1], gath[ch + 1] = start_gather(ch + 1)
    scat[_NCH - 1].wait()


_shuffle = functools.partial(
    pl.kernel,
    mesh=plsc.VectorSubcoreMesh(core_axis_name="c", subcore_axis_name="s"),
    out_type=jax.ShapeDtypeStruct((ROWS, HW), jnp.float32),
    scratch_types=[
        pltpu.VMEM((C,), jnp.int32),
        pltpu.VMEM((2, CHUNK), jnp.int32),
        pltpu.VMEM((2, CHUNK, HW), jnp.float32),
        pltpu.SemaphoreType.DMA,
        pltpu.SemaphoreType.DMA,
        pltpu.SemaphoreType.DMA,
        pltpu.SemaphoreType.DMA,
    ],
)(_shuffle_body)


def kernel(x, objective, indices):
    y = _shuffle(x.reshape(ROWS, HW), indices)
    return y.reshape(B, C, H, W), objective


# speedup vs baseline: 1.0020x; 1.0006x over previous
import functools

import jax
import jax.numpy as jnp
from jax import lax
from jax.experimental import pallas as pl
from jax.experimental.pallas import tpu as pltpu
from jax.experimental.pallas import tpu_sc as plsc

B, C, H, W = 64, 192, 32, 32
ROWS = B * C

_info = plsc.get_sparse_core_info()
_NC, _NS, _L = _info.num_cores, _info.num_subcores, _info.num_lanes
_NW = _NC * _NS
_BPW = B // _NW
CHUNK = 48
_CPB = C // CHUNK
_NCH = _BPW * _CPB


def _shuffle_body(x_hbm, perm_hbm, out_hbm, perm_v, idx_v, rows_v,
                  gs0, gs1, ss0, ss1):
    wid = lax.axis_index("s") * _NC + lax.axis_index("c")
    pltpu.sync_copy(perm_hbm, perm_v)
    gsem = (gs0, gs1)
    ssem = (ss0, ss1)

    def start_gather(ch):
        buf = ch % 2
        bi, c0i = divmod(ch, _CPB)
        c0 = c0i * CHUNK
        bbase = (wid * _BPW + bi) * C
        for i in range(CHUNK // _L):
            idx_v[buf, pl.ds(i * _L, _L)] = (
                perm_v[pl.ds(c0 + i * _L, _L)] + bbase)
        cp = pltpu.async_copy(x_hbm.at[idx_v.at[buf]], rows_v.at[buf],
                              gsem[buf])
        return bbase + c0, cp

    offs = [None] * _NCH
    gath = [None] * _NCH
    scat = [None] * _NCH
    offs[0], gath[0] = start_gather(0)
    for ch in range(_NCH):
        buf = ch % 2
        gath[ch].wait()
        scat[ch] = pltpu.async_copy(
            rows_v.at[buf], out_hbm.at[pl.ds(offs[ch], CHUNK)], ssem[buf])
        if ch + 1 < _NCH:
            if ch >= 1:
                scat[ch - 1].wait()
            offs[ch + 1], gath[ch + 1] = start_gather(ch + 1)
    scat[_NCH - 1].wait()


_shuffle = functools.partial(
    pl.kernel,
    mesh=plsc.VectorSubcoreMesh(core_axis_name="c", subcore_axis_name="s"),
    out_type=jax.ShapeDtypeStruct((ROWS, 8, 128), jnp.float32),
    scratch_types=[
        pltpu.VMEM((C,), jnp.int32),
        pltpu.VMEM((2, CHUNK), jnp.int32),
        pltpu.VMEM((2, CHUNK, 8, 128), jnp.float32),
        pltpu.SemaphoreType.DMA,
        pltpu.SemaphoreType.DMA,
        pltpu.SemaphoreType.DMA,
        pltpu.SemaphoreType.DMA,
    ],
)(_shuffle_body)


def kernel(x, objective, indices):
    y = _shuffle(x.reshape(ROWS, 8, 128), indices)
    return y.reshape(B, C, H, W), objective


# speedup vs baseline: 2.3211x; 2.3164x over previous
import functools

import jax
import jax.numpy as jnp
from jax import lax
from jax.experimental import pallas as pl
from jax.experimental.pallas import tpu as pltpu
from jax.experimental.pallas import tpu_sc as plsc

B, C, H, W = 64, 192, 32, 32
P = B * H * W

_info = plsc.get_sparse_core_info()
_NC, _NS, _L = _info.num_cores, _info.num_subcores, _info.num_lanes
_NW = _NC * _NS
_PPW = P // _NW
PB = 64
_NBLK = _PPW // PB
_NG = C // _L


def _body(x_hbm, perm_hbm, out_hbm, perm_v, xbuf, obuf, gs0, gs1, ss0, ss1):
    wid = lax.axis_index("s") * _NC + lax.axis_index("c")
    base = wid * _PPW
    pltpu.sync_copy(perm_hbm, perm_v)
    perms = tuple(perm_v[pl.ds(g * _L, _L)] for g in range(_NG))
    gsem = (gs0, gs1)
    ssem = (ss0, ss1)

    def dma_in(blk, buf):
        return pltpu.async_copy(x_hbm.at[pl.ds(base + blk * PB, PB)],
                                xbuf.at[buf], gsem[buf])

    def dma_out(blk, buf):
        return pltpu.async_copy(obuf.at[buf],
                                out_hbm.at[pl.ds(base + blk * PB, PB)],
                                ssem[buf])

    gath = {0: dma_in(0, 0)}
    scat = {}
    for blk in range(_NBLK):
        buf = blk % 2
        if blk + 1 < _NBLK:
            if blk >= 1:
                scat[blk - 1].wait()
            gath[blk + 1] = dma_in(blk + 1, 1 - buf)
        gath[blk].wait()

        def step(p, carry):
            for g in range(_NG):
                row = jnp.full((_L,), p, jnp.int32)
                vals = plsc.load_gather(xbuf.at[buf], [row, carry[g]])
                obuf[buf, p, pl.ds(g * _L, _L)] = vals
            return carry

        lax.fori_loop(0, PB, step, perms)
        scat[blk] = dma_out(blk, buf)
    scat[_NBLK - 2].wait()
    scat[_NBLK - 1].wait()


_shuffle = functools.partial(
    pl.kernel,
    mesh=plsc.VectorSubcoreMesh(core_axis_name="c", subcore_axis_name="s"),
    out_type=jax.ShapeDtypeStruct((P, C), jnp.float32),
    scratch_types=[
        pltpu.VMEM((C,), jnp.int32),
        pltpu.VMEM((2, PB, C), jnp.float32),
        pltpu.VMEM((2, PB, C), jnp.float32),
        pltpu.SemaphoreType.DMA,
        pltpu.SemaphoreType.DMA,
        pltpu.SemaphoreType.DMA,
        pltpu.SemaphoreType.DMA,
    ],
    compiler_params=pltpu.CompilerParams(use_tc_tiling_on_sc=True,
                                         needs_layout_passes=False),
)(_body)


def kernel(x, objective, indices):
    x2 = jnp.transpose(x, (0, 2, 3, 1)).reshape(P, C)
    y2 = _shuffle(x2, indices)
    y = jnp.transpose(y2.reshape(B, H, W, C), (0, 3, 1, 2))
    return y, objective


# speedup vs baseline: 4.4608x; 1.9218x over previous
import functools

import jax
import jax.numpy as jnp
from jax import lax
from jax.experimental import pallas as pl
from jax.experimental.pallas import tpu as pltpu
from jax.experimental.pallas import tpu_sc as plsc

B, C, H, W = 64, 192, 32, 32
P = B * H * W

_info = plsc.get_sparse_core_info()
_NC, _NS, _L = _info.num_cores, _info.num_subcores, _info.num_lanes
_NW = _NC * _NS
_PPW = P // _NW
PB = 64
_NBLK = _PPW // PB
_NG = C // _L


def _body(x_hbm, perm_hbm, out_hbm, perm_v, xbuf, obuf, gs0, gs1, ss0, ss1):
    wid = lax.axis_index("s") * _NC + lax.axis_index("c")
    base = wid * _PPW
    pltpu.sync_copy(perm_hbm, perm_v)
    perms = tuple(perm_v[pl.ds(g * _L, _L)] for g in range(_NG))
    gsem = (gs0, gs1)
    ssem = (ss0, ss1)

    def dma_in(blk, buf):
        return pltpu.async_copy(x_hbm.at[pl.ds(base + blk * PB, PB)],
                                xbuf.at[buf], gsem[buf])

    def dma_out(blk, buf):
        return pltpu.async_copy(obuf.at[buf],
                                out_hbm.at[pl.ds(base + blk * PB, PB)],
                                ssem[buf])

    gath = {0: dma_in(0, 0)}
    scat = {}
    for blk in range(_NBLK):
        buf = blk % 2
        if blk + 1 < _NBLK:
            if blk >= 1:
                scat[blk - 1].wait()
            gath[blk + 1] = dma_in(blk + 1, 1 - buf)
        gath[blk].wait()

        @plsc.parallel_loop(0, PB, unroll=2, carry=perms)
        def step(p, carry):
            row = jnp.full((_L,), p, jnp.int32)
            vals = [plsc.load_gather(xbuf.at[buf], [row, carry[g]])
                    for g in range(_NG)]
            for g in range(_NG):
                obuf[buf, p, pl.ds(g * _L, _L)] = vals[g]
            return carry

        scat[blk] = dma_out(blk, buf)
    scat[_NBLK - 2].wait()
    scat[_NBLK - 1].wait()


_shuffle = functools.partial(
    pl.kernel,
    mesh=plsc.VectorSubcoreMesh(core_axis_name="c", subcore_axis_name="s"),
    out_type=jax.ShapeDtypeStruct((P, C), jnp.float32),
    scratch_types=[
        pltpu.VMEM((C,), jnp.int32),
        pltpu.VMEM((2, PB, C), jnp.float32),
        pltpu.VMEM((2, PB, C), jnp.float32),
        pltpu.SemaphoreType.DMA,
        pltpu.SemaphoreType.DMA,
        pltpu.SemaphoreType.DMA,
        pltpu.SemaphoreType.DMA,
    ],
    compiler_params=pltpu.CompilerParams(use_tc_tiling_on_sc=True,
                                         needs_layout_passes=False),
)(_body)


def kernel(x, objective, indices):
    x2 = jnp.transpose(x, (0, 2, 3, 1)).reshape(P, C)
    y2 = _shuffle(x2, indices)
    y = jnp.transpose(y2.reshape(B, H, W, C), (0, 3, 1, 2))
    return y, objective


# speedup vs baseline: 4.7878x; 1.0733x over previous
import functools

import jax
import jax.numpy as jnp
from jax import lax
from jax.experimental import pallas as pl
from jax.experimental.pallas import tpu as pltpu
from jax.experimental.pallas import tpu_sc as plsc

B, C, H, W = 64, 192, 32, 32
P = B * H * W

_info = plsc.get_sparse_core_info()
_NC, _NS, _L = _info.num_cores, _info.num_subcores, _info.num_lanes
_NW = _NC * _NS
_PPW = P // _NW
PB = 64
_NBLK = _PPW // PB
_NG = C // _L
_NBUF = 3


def _body(x_hbm, perm_hbm, out_hbm, perm_v, xbuf, obuf,
          g0, g1, g2, s0, s1, s2):
    wid = lax.axis_index("s") * _NC + lax.axis_index("c")
    base = wid * _PPW
    pltpu.sync_copy(perm_hbm, perm_v)
    perms = tuple(perm_v[pl.ds(g * _L, _L)] for g in range(_NG))
    gsem = (g0, g1, g2)
    ssem = (s0, s1, s2)

    def dma_in(blk):
        i = blk % _NBUF
        return pltpu.async_copy(x_hbm.at[pl.ds(base + blk * PB, PB)],
                                xbuf.at[i], gsem[i])

    def dma_out(blk):
        i = blk % _NBUF
        return pltpu.async_copy(obuf.at[i],
                                out_hbm.at[pl.ds(base + blk * PB, PB)],
                                ssem[i])

    gath = {}
    scat = {}
    for k in range(min(_NBUF, _NBLK)):
        gath[k] = dma_in(k)
    for blk in range(_NBLK):
        i = blk % _NBUF
        gath[blk].wait()
        if blk >= _NBUF:
            scat[blk - _NBUF].wait()

        @plsc.parallel_loop(0, PB, unroll=2, carry=perms)
        def step(p, carry):
            row = jnp.full((_L,), p, jnp.int32)
            vals = [plsc.load_gather(xbuf.at[i], [row, carry[g]])
                    for g in range(_NG)]
            for g in range(_NG):
                obuf[i, p, pl.ds(g * _L, _L)] = vals[g]
            return carry

        scat[blk] = dma_out(blk)
        if blk + _NBUF < _NBLK:
            gath[blk + _NBUF] = dma_in(blk + _NBUF)
    for k in range(max(0, _NBLK - _NBUF), _NBLK):
        scat[k].wait()


_shuffle = functools.partial(
    pl.kernel,
    mesh=plsc.VectorSubcoreMesh(core_axis_name="c", subcore_axis_name="s"),
    out_type=jax.ShapeDtypeStruct((P, C), jnp.float32),
    scratch_types=[
        pltpu.VMEM((C,), jnp.int32),
        pltpu.VMEM((_NBUF, PB, C), jnp.float32),
        pltpu.VMEM((_NBUF, PB, C), jnp.float32),
        pltpu.SemaphoreType.DMA,
        pltpu.SemaphoreType.DMA,
        pltpu.SemaphoreType.DMA,
        pltpu.SemaphoreType.DMA,
        pltpu.SemaphoreType.DMA,
        pltpu.SemaphoreType.DMA,
    ],
    compiler_params=pltpu.CompilerParams(use_tc_tiling_on_sc=True,
                                         needs_layout_passes=False),
)(_body)


def kernel(x, objective, indices):
    x2 = jnp.transpose(x, (0, 2, 3, 1)).reshape(P, C)
    y2 = _shuffle(x2, indices)
    y = jnp.transpose(y2.reshape(B, H, W, C), (0, 3, 1, 2))
    return y, objective
